# baseline (speedup 1.0000x reference)
import jax
import jax.numpy as jnp
from jax.experimental import pallas as pl
from jax.experimental.pallas import tpu as pltpu

_N_CHUNKS = 8


def _copy_body(x_hbm, o_hbm, sem):
    B = x_hbm.shape[0]
    rows = B // _N_CHUNKS
    for c in range(_N_CHUNKS):
        pltpu.make_async_copy(
            x_hbm.at[pl.ds(c * rows, rows), :],
            o_hbm.at[pl.ds(c * rows, rows), :],
            sem,
        ).start()
    for c in range(_N_CHUNKS):
        pltpu.make_async_copy(
            x_hbm.at[pl.ds(c * rows, rows), :],
            o_hbm.at[pl.ds(c * rows, rows), :],
            sem,
        ).wait()


def kernel(logits, attention_mask):
    B, V = logits.shape
    out = pl.pallas_call(
        _copy_body,
        in_specs=[pl.BlockSpec(memory_space=pltpu.MemorySpace.HBM)],
        out_specs=pl.BlockSpec(memory_space=pltpu.MemorySpace.HBM),
        out_shape=jax.ShapeDtypeStruct((B, V), jnp.float32),
        scratch_shapes=[pltpu.SemaphoreType.DMA],
    )(logits)
    return out


# speedup vs baseline: 43.5203x; 43.5203x over previous
import jax
import jax.numpy as jnp
from jax.experimental import pallas as pl
from jax.experimental.pallas import tpu as pltpu

_BR = 8


def _fill_body(x_ref, o_ref):
    o_ref[...] = x_ref[...]


def kernel(logits, attention_mask):
    B, V = logits.shape
    out = pl.pallas_call(
        _fill_body,
        grid=(B // _BR,),
        in_specs=[pl.BlockSpec((_BR, V), lambda i: (i, 0))],
        out_specs=pl.BlockSpec((_BR, V), lambda i: (i, 0)),
        out_shape=jax.ShapeDtypeStruct((B, V), jnp.float32),
        compiler_params=pltpu.CompilerParams(
            dimension_semantics=("arbitrary",),
        ),
    )(logits)
    return out
